# baseline (speedup 1.0000x reference)
import dataclasses
import functools

import jax
import jax.numpy as jnp
from jax import lax
from jax.experimental import pallas as pl
from jax.experimental.pallas import tpu as pltpu
from jax.experimental.pallas import tpu_sc as plsc

N = 10000
NP = 10240
E = 320000
F = 128

NC = 2
NS = 16
NW = NC * NS
EPT = E // NW
CH = 40
NCH = EPT // CH
EPT2 = 10240
RPT = NP // NS
NGRP = NCH // 4

_mesh = plsc.VectorSubcoreMesh(core_axis_name="c", subcore_axis_name="s")

_cp = pltpu.CompilerParams()
if "needs_layout_passes" in pltpu.CompilerParams.__dataclass_fields__:
    _cp = dataclasses.replace(_cp, needs_layout_passes=False)


@functools.partial(
    pl.kernel,
    out_type=jax.ShapeDtypeStruct((NW * NP,), jnp.float32),
    mesh=_mesh,
    compiler_params=_cp,
    scratch_types=[
        pltpu.VMEM((EPT2,), jnp.int32),
        pltpu.VMEM((NP,), jnp.float32),
    ],
)
def _deg_sc(dst_hbm, out_hbm, idx_v, hist_v):
    c = lax.axis_index("c")
    s = lax.axis_index("s")
    wid = c * NS + s
    pltpu.sync_copy(dst_hbm.at[wid], idx_v)

    @pl.loop(0, NP, step=16)
    def _(i):
        hist_v[pl.ds(i, 16)] = jnp.zeros((16,), jnp.float32)

    ones = jnp.ones((16,), jnp.float32)

    @pl.loop(0, EPT2, step=16)
    def _(j):
        plsc.addupdate_scatter(hist_v, [idx_v[pl.ds(j, 16)]], ones)

    pltpu.sync_copy(hist_v, out_hbm.at[pl.ds(wid * NP, NP)])


@functools.partial(
    pl.kernel,
    out_type=jax.ShapeDtypeStruct((NC * NP, F), jnp.float32),
    mesh=_mesh,
    scratch_types=[
        pltpu.VMEM((CH,), jnp.int32),
        pltpu.VMEM((CH,), jnp.int32),
        pltpu.VMEM((CH, F), jnp.float32),
        pltpu.VMEM_SHARED((NP, F), jnp.float32),
    ],
)
def _agg_sc(h_hbm, src_hbm, dst_hbm, out_hbm, src_v, dst_v, rows_v, acc_sh):
    c = lax.axis_index("c")
    s = lax.axis_index("s")
    wid = c * NS + s
    rbase = s * RPT
    pltpu.sync_copy(h_hbm.at[pl.ds(rbase, RPT)], acc_sh.at[pl.ds(rbase, RPT)])
    plsc.subcore_barrier()
    ebase = wid * EPT

    @pl.loop(0, NCH)
    def _(k):
        pltpu.sync_copy(src_hbm.at[pl.ds(ebase + k * CH, CH)], src_v)
        pltpu.sync_copy(dst_hbm.at[pl.ds(ebase + k * CH, CH)], dst_v)
        pltpu.sync_copy(h_hbm.at[src_v], rows_v)
        pltpu.sync_copy(rows_v, acc_sh.at[dst_v], add=True)

    plsc.subcore_barrier()
    pltpu.sync_copy(acc_sh.at[pl.ds(rbase, RPT)],
                    out_hbm.at[pl.ds(c * NP + rbase, RPT)])


def _dis_body(h_ref, out_ref):
    ones = jnp.ones((NW, 1), jnp.float32)
    deg = lax.dot_general(h_ref[...], ones, (((0,), (0,)), ((), ())),
                          precision=lax.Precision.HIGHEST,
                          preferred_element_type=jnp.float32) + 1.0
    out_ref[...] = jnp.broadcast_to(lax.rsqrt(deg), out_ref.shape)


_dis_tc = pl.pallas_call(
    _dis_body,
    grid=(NP // 128,),
    in_specs=[pl.BlockSpec((NW, 128), lambda i: (0, i))],
    out_specs=pl.BlockSpec((128, 128), lambda i: (i, 0)),
    out_shape=jax.ShapeDtypeStruct((NP, 128), jnp.float32),
)

BM = 512


def _mm_body(x_ref, w_ref, d_ref, out_ref):
    h = jnp.dot(x_ref[...], w_ref[...], preferred_element_type=jnp.float32,
                precision=lax.Precision.HIGHEST)
    out_ref[...] = h * d_ref[...]


_mm_tc = pl.pallas_call(
    _mm_body,
    grid=(NP // BM,),
    in_specs=[pl.BlockSpec((BM, F), lambda i: (i, 0)),
              pl.BlockSpec((F, F), lambda i: (0, 0)),
              pl.BlockSpec((BM, F), lambda i: (i, 0))],
    out_specs=pl.BlockSpec((BM, F), lambda i: (i, 0)),
    out_shape=jax.ShapeDtypeStruct((NP, F), jnp.float32),
)


def _fin_body(a0_ref, a1_ref, hp_ref, d_ref, b_ref, out_ref, *, relu):
    v = (a0_ref[...] + a1_ref[...] - hp_ref[...]) * d_ref[...] + b_ref[...]
    out_ref[...] = jnp.maximum(v, 0.0) if relu else v


def _make_fin(relu):
    return pl.pallas_call(
        functools.partial(_fin_body, relu=relu),
        grid=(NP // BM,),
        in_specs=[pl.BlockSpec((BM, F), lambda i: (i, 0)),
                  pl.BlockSpec((BM, F), lambda i: (i + NP // BM, 0)),
                  pl.BlockSpec((BM, F), lambda i: (i, 0)),
                  pl.BlockSpec((BM, F), lambda i: (i, 0)),
                  pl.BlockSpec((1, F), lambda i: (0, 0))],
        out_specs=pl.BlockSpec((BM, F), lambda i: (i, 0)),
        out_shape=jax.ShapeDtypeStruct((NP, F), jnp.float32),
    )


_fin_relu_tc = _make_fin(True)
_fin_tc = _make_fin(False)


def kernel(x, edge_index, W1, b1, W2, b2):
    src = edge_index[0].astype(jnp.int32).reshape(NW, EPT)
    dst = edge_index[1].astype(jnp.int32).reshape(NW, EPT)
    pad = jnp.full((NW, EPT2 - EPT), NP - 1, jnp.int32)
    srcp = jnp.concatenate([src, pad], axis=1)
    dstp = jnp.concatenate([dst, pad], axis=1)
    xp = jnp.zeros((NP, F), jnp.float32).at[:N].set(x)
    b1r = b1.reshape(1, F)
    b2r = b2.reshape(1, F)

    hist = _deg_sc(dstp).reshape(NW, NP)
    dismat = _dis_tc(hist)

    h1 = _mm_tc(xp, W1, dismat)
    acc1 = _agg_sc(h1, src.reshape(-1), dst.reshape(-1))
    g1 = _fin_relu_tc(acc1, acc1, h1, dismat, b1r)

    h2 = _mm_tc(g1, W2, dismat)
    acc2 = _agg_sc(h2, src.reshape(-1), dst.reshape(-1))
    out = _fin_tc(acc2, acc2, h2, dismat, b2r)
    return out[:N]


# speedup vs baseline: 1.6522x; 1.6522x over previous
import dataclasses
import functools

import jax
import jax.numpy as jnp
from jax import lax
from jax.experimental import pallas as pl
from jax.experimental.pallas import tpu as pltpu
from jax.experimental.pallas import tpu_sc as plsc

N = 10000
NP = 10240
E = 320000
F = 128

NC = 2
NS = 16
NW = NC * NS
EPT = E // NW
CH = 80
NCH = EPT // CH
EPT2 = 10240
RPT = NP // NS
NGRP = NCH // 4

_mesh = plsc.VectorSubcoreMesh(core_axis_name="c", subcore_axis_name="s")

_cp = pltpu.CompilerParams()
if "needs_layout_passes" in pltpu.CompilerParams.__dataclass_fields__:
    _cp = dataclasses.replace(_cp, needs_layout_passes=False)


@functools.partial(
    pl.kernel,
    out_type=jax.ShapeDtypeStruct((NW * NP,), jnp.float32),
    mesh=_mesh,
    compiler_params=_cp,
    scratch_types=[
        pltpu.VMEM((EPT2,), jnp.int32),
        pltpu.VMEM((NP,), jnp.float32),
    ],
)
def _deg_sc(dst_hbm, out_hbm, idx_v, hist_v):
    c = lax.axis_index("c")
    s = lax.axis_index("s")
    wid = c * NS + s
    pltpu.sync_copy(dst_hbm.at[wid], idx_v)

    @pl.loop(0, NP, step=16)
    def _(i):
        hist_v[pl.ds(i, 16)] = jnp.zeros((16,), jnp.float32)

    ones = jnp.ones((16,), jnp.float32)

    @pl.loop(0, EPT2, step=16)
    def _(j):
        plsc.addupdate_scatter(hist_v, [idx_v[pl.ds(j, 16)]], ones)

    pltpu.sync_copy(hist_v, out_hbm.at[pl.ds(wid * NP, NP)])


@functools.partial(
    pl.kernel,
    out_type=jax.ShapeDtypeStruct((NC * NP, F), jnp.float32),
    mesh=_mesh,
    scratch_types=[
        pltpu.VMEM((2, CH), jnp.int32),
        pltpu.VMEM((CH, F), jnp.float32),
        pltpu.VMEM_SHARED((NP, F), jnp.float32),
    ],
)
def _agg_sc(h_hbm, eidx_hbm, out_hbm, eidx_v, rows_v, acc_sh):
    c = lax.axis_index("c")
    s = lax.axis_index("s")
    wid = c * NS + s
    rbase = s * RPT
    pltpu.sync_copy(h_hbm.at[pl.ds(rbase, RPT)], acc_sh.at[pl.ds(rbase, RPT)])
    plsc.subcore_barrier()
    cbase = wid * NCH

    @pl.loop(0, NCH)
    def _(k):
        pltpu.sync_copy(eidx_hbm.at[k + cbase], eidx_v)
        pltpu.sync_copy(h_hbm.at[eidx_v.at[0]], rows_v)
        pltpu.sync_copy(rows_v, acc_sh.at[eidx_v.at[1]],
                        add=True)

    plsc.subcore_barrier()
    pltpu.sync_copy(acc_sh.at[pl.ds(rbase, RPT)],
                    out_hbm.at[pl.ds(c * NP + rbase, RPT)])


def _dis_body(h_ref, out_ref):
    ones = jnp.ones((NW, 1), jnp.float32)
    deg = lax.dot_general(h_ref[...], ones, (((0,), (0,)), ((), ())),
                          precision=lax.Precision.HIGHEST,
                          preferred_element_type=jnp.float32) + 1.0
    out_ref[...] = jnp.broadcast_to(lax.rsqrt(deg), out_ref.shape)


_dis_tc = pl.pallas_call(
    _dis_body,
    grid=(NP // 128,),
    in_specs=[pl.BlockSpec((NW, 128), lambda i: (0, i))],
    out_specs=pl.BlockSpec((128, 128), lambda i: (i, 0)),
    out_shape=jax.ShapeDtypeStruct((NP, 128), jnp.float32),
)

BM = 512


def _mm_body(x_ref, w_ref, d_ref, out_ref):
    h = jnp.dot(x_ref[...], w_ref[...], preferred_element_type=jnp.float32,
                precision=lax.Precision.HIGHEST)
    out_ref[...] = h * d_ref[...]


_mm_tc = pl.pallas_call(
    _mm_body,
    grid=(NP // BM,),
    in_specs=[pl.BlockSpec((BM, F), lambda i: (i, 0)),
              pl.BlockSpec((F, F), lambda i: (0, 0)),
              pl.BlockSpec((BM, F), lambda i: (i, 0))],
    out_specs=pl.BlockSpec((BM, F), lambda i: (i, 0)),
    out_shape=jax.ShapeDtypeStruct((NP, F), jnp.float32),
)


def _fin_body(a0_ref, a1_ref, hp_ref, d_ref, b_ref, out_ref, *, relu):
    v = (a0_ref[...] + a1_ref[...] - hp_ref[...]) * d_ref[...] + b_ref[...]
    out_ref[...] = jnp.maximum(v, 0.0) if relu else v


def _make_fin(relu):
    return pl.pallas_call(
        functools.partial(_fin_body, relu=relu),
        grid=(NP // BM,),
        in_specs=[pl.BlockSpec((BM, F), lambda i: (i, 0)),
                  pl.BlockSpec((BM, F), lambda i: (i + NP // BM, 0)),
                  pl.BlockSpec((BM, F), lambda i: (i, 0)),
                  pl.BlockSpec((BM, F), lambda i: (i, 0)),
                  pl.BlockSpec((1, F), lambda i: (0, 0))],
        out_specs=pl.BlockSpec((BM, F), lambda i: (i, 0)),
        out_shape=jax.ShapeDtypeStruct((NP, F), jnp.float32),
    )


_fin_relu_tc = _make_fin(True)
_fin_tc = _make_fin(False)


def kernel(x, edge_index, W1, b1, W2, b2):
    src = edge_index[0].astype(jnp.int32).reshape(NW, EPT)
    dst = edge_index[1].astype(jnp.int32).reshape(NW, EPT)
    pad = jnp.full((NW, EPT2 - EPT), NP - 1, jnp.int32)
    srcp = jnp.concatenate([src, pad], axis=1)
    dstp = jnp.concatenate([dst, pad], axis=1)
    xp = jnp.zeros((NP, F), jnp.float32).at[:N].set(x)
    b1r = b1.reshape(1, F)
    b2r = b2.reshape(1, F)

    ec = jnp.stack([src.reshape(NW, NCH, CH), dst.reshape(NW, NCH, CH)],
                   axis=2).reshape(NW * NCH, 2, CH)
    hist = _deg_sc(dstp).reshape(NW, NP)
    dismat = _dis_tc(hist)

    h1 = _mm_tc(xp, W1, dismat)
    acc1 = _agg_sc(h1, ec)
    g1 = _fin_relu_tc(acc1, acc1, h1, dismat, b1r)

    h2 = _mm_tc(g1, W2, dismat)
    acc2 = _agg_sc(h2, ec)
    out = _fin_tc(acc2, acc2, h2, dismat, b2r)
    return out[:N]
